# baseline (speedup 1.0000x reference)
import functools

import jax
import jax.numpy as jnp
from jax.experimental import pallas as pl


def _mm_bias_kernel(x_ref, w_ref, b_ref, o_ref):
    o_ref[...] = (
        jnp.dot(x_ref[...], w_ref[...], preferred_element_type=jnp.float32)
        + b_ref[...][None, :]
    )


def _mm_bias(x, w, b, block_rows=None):
    m, k = x.shape
    n = w.shape[1]
    if block_rows is None:
        block_rows = m
    grid = (m // block_rows,)
    return pl.pallas_call(
        _mm_bias_kernel,
        grid=grid,
        in_specs=[
            pl.BlockSpec((block_rows, k), lambda i: (i, 0)),
            pl.BlockSpec((k, n), lambda i: (0, 0)),
            pl.BlockSpec((n,), lambda i: (0,)),
        ],
        out_specs=pl.BlockSpec((block_rows, n), lambda i: (i, 0)),
        out_shape=jax.ShapeDtypeStruct((m, n), jnp.float32),
    )(x, w, b)


def _gat_layer(x, edge_index, edge_attr, W_l, b_l, W_r, b_r, W_e, att, bias,
               heads, outc, concat, neg_slope, share):
    N = x.shape[0]
    src = edge_index[0]
    dst = edge_index[1]
    x_l = _mm_bias(x, W_l, b_l).reshape(-1, heads, outc)
    if share:
        x_r = x_l
    else:
        x_r = _mm_bias(x, W_r, b_r).reshape(-1, heads, outc)
    e_feat = _mm_bias(edge_attr, W_e, jnp.zeros((W_e.shape[1],), jnp.float32),
                      block_rows=8000).reshape(-1, heads, outc)
    e = x_l[src] + x_r[dst] + e_feat
    e = jnp.where(e > 0, e, neg_slope * e)
    alpha = jnp.sum(e * att[None, :, :], axis=-1)
    amax = jax.ops.segment_max(alpha, dst, num_segments=N)
    amax = jnp.where(jnp.isfinite(amax), amax, 0.0)
    ex = jnp.exp(alpha - amax[dst])
    den = jax.ops.segment_sum(ex, dst, num_segments=N)
    a = ex / (den[dst] + 1e-16)
    out = jax.ops.segment_sum(x_l[src] * a[:, :, None], dst, num_segments=N)
    if concat:
        out = out.reshape(N, heads * outc)
    else:
        out = out.mean(axis=1)
    return out + bias


def kernel(x, edge_index, edge_attr, W1, b1, We1, att1, bias1, g1, be1,
           W2, b2, We2, att2, bias2, g2, be2,
           W3l, b3l, W3r, b3r, We3, att3, bias3):
    h = _gat_layer(x, edge_index, edge_attr, W1, b1, W1, b1, We1, att1, bias1,
                   8, 32, True, 0.2, True)
    h = g1 * h / jnp.sqrt(1.0 + 1e-5) + be1
    h = jnp.where(h > 0, h, jnp.expm1(h))
    h = _gat_layer(h, edge_index, edge_attr, W2, b2, W2, b2, We2, att2, bias2,
                   8, 16, True, 0.2, True)
    h = g2 * h / jnp.sqrt(1.0 + 1e-5) + be2
    h = jnp.where(h > 0, h, jnp.expm1(h))
    out = _gat_layer(h, edge_index, edge_attr, W3l, b3l, W3r, b3r, We3, att3,
                     bias3, 1, 1, False, 0.2, False)
    return out


# speedup vs baseline: 14.9953x; 14.9953x over previous
import functools

import jax
import jax.numpy as jnp
from jax import lax
from jax.experimental import pallas as pl
from jax.experimental.pallas import tpu as pltpu
from jax.experimental.pallas import tpu_sc as plsc

NC = 2
NS = 16
L = 16
BLK = 128


def _perm_cols_split(a, K):
    lead = a.shape[:-1]
    r = a.reshape(*lead, 2, 4, 4, K)
    r = jnp.moveaxis(r, -1, -3)
    return r.reshape(*lead, 2, 16 * K)


def _perm_cols_flat(a, K):
    lead = a.shape[:-1]
    r = a.reshape(*lead, 8, 2, K)
    r = jnp.moveaxis(r, -1, -3)
    return r.reshape(*lead, 16 * K)


def _sc_mesh():
    return plsc.VectorSubcoreMesh(core_axis_name="c", subcore_axis_name="s",
                                  num_cores=NC, num_subcores=NS)


def _zero_wbd(wbd, iot):
    def zrow(t, carry):
        cols = (t & 7) * L + iot
        row = jnp.full((L,), t >> 3, jnp.int32)
        plsc.store_scatter(wbd, [row, cols], jnp.zeros((L,), jnp.float32))
        return carry
    lax.fori_loop(0, BLK * 8, zrow, 0)


def _pass_a(u2, ef2, src, dstg, dsts, attc, zd, u_split):
    K = 8
    n_nodes = (u2.shape[1] if u_split else u2.shape[0])
    n_edges = src.shape[0]
    rows_pt = (-(-n_nodes // NS) + 7) // 8 * 8
    n_pad = rows_pt * NS
    rowsd_pt = (-(-(n_pad // 8) // NS) + 7) // 8 * 8
    nd_pad = rowsd_pt * NS
    nworkers = NS if u_split else NC * NS
    ept = n_edges // nworkers
    nblk = ept // BLK
    nshuf = 2 if u_split else 1

    def body(u_ref, ef_ref, src_ref, dstg_ref, dsts_ref, att_ref, zd_ref,
             ox_ref, od_ref,
             sidx, didxg, didxs, didxp, didx8, us, ud, ef, wbd, exb, attv,
             accd, sem1, sem2):
        c = lax.axis_index("c")
        s = lax.axis_index("s")
        pltpu.sync_copy(att_ref.at[c] if u_split else att_ref, attv)
        rd0 = s * rowsd_pt
        pltpu.sync_copy(zd_ref, accd.at[pl.ds(rd0, rowsd_pt)])
        iot = lax.iota(jnp.int32, L)
        _zero_wbd(wbd, iot)
        plsc.subcore_barrier()
        shufs = [iot ^ 1, iot ^ 2][:nshuf]
        w = s if u_split else (c * NS + s)

        def blk_body(b, carry):
            base = w * ept + b * BLK
            pltpu.sync_copy(src_ref.at[pl.ds(base, BLK)], sidx)
            pltpu.sync_copy(dstg_ref.at[pl.ds(base, BLK)], didxg)
            pltpu.sync_copy(dsts_ref.at[pl.ds(base, BLK)], didxs)
            pltpu.sync_copy(dsts_ref.at[pl.ds(base, BLK)],
                            didxp.at[pl.ds(0, BLK)])
            if u_split:
                cp1 = pltpu.async_copy(u_ref.at[c].at[sidx], us, sem1)
                cp2 = pltpu.async_copy(u_ref.at[c].at[didxg], ud, sem2)
                pltpu.sync_copy(ef_ref.at[c].at[pl.ds(base, BLK)], ef)
            else:
                cp1 = pltpu.async_copy(u_ref.at[sidx], us, sem1)
                cp2 = pltpu.async_copy(u_ref.at[didxg], ud, sem2)
                pltpu.sync_copy(ef_ref.at[pl.ds(base, BLK)], ef)
            for j in range(BLK // L):
                sl = pl.ds(j * L, L)
                didx8[sl] = jnp.right_shift(didxs[sl], 3)
            cp1.wait()
            cp2.wait()

            def edge(i, carry2):
                accv = jnp.zeros((L,), jnp.float32)
                for k in range(K):
                    sl = pl.ds(k * L, L)
                    e = us[i, sl] + ud[i, sl] + ef[i, sl]
                    e = jnp.maximum(e, 0.0) + 0.2 * jnp.minimum(e, 0.0)
                    accv = accv + e * attv[k]
                for sh in shufs:
                    accv = accv + jnp.take_along_axis(accv, sh, axis=0)
                exv = jnp.exp(jnp.minimum(accv, 60.0))
                exb[i, :] = exv
                d = didxp[pl.ds(i, L)][0]
                cols = (d & 7) * L + iot
                row = jnp.full((L,), i, jnp.int32)
                plsc.store_scatter(wbd, [row, cols], exv)
                return carry2

            lax.fori_loop(0, BLK, edge, 0)
            pltpu.sync_copy(wbd, accd.at[didx8], add=True)
            if u_split:
                pltpu.sync_copy(exb, ox_ref.at[c].at[pl.ds(base, BLK)])
            else:
                pltpu.sync_copy(exb, ox_ref.at[0].at[pl.ds(base, BLK)])

            def zedge(i, carry2):
                d = didxp[pl.ds(i, L)][0]
                cols = (d & 7) * L + iot
                row = jnp.full((L,), i, jnp.int32)
                plsc.store_scatter(wbd, [row, cols],
                                   jnp.zeros((L,), jnp.float32))
                return carry2

            lax.fori_loop(0, BLK, zedge, 0)
            return carry

        lax.fori_loop(0, nblk, blk_body, 0)
        plsc.subcore_barrier()
        pltpu.sync_copy(accd.at[pl.ds(rd0, rowsd_pt)],
                        od_ref.at[c].at[pl.ds(rd0, rowsd_pt)])

    n_ex = NC if u_split else 1
    f = pl.kernel(
        body,
        out_type=(
            jax.ShapeDtypeStruct((n_ex, n_edges, L), jnp.float32),
            jax.ShapeDtypeStruct((NC, nd_pad, 128), jnp.float32),
        ),
        mesh=_sc_mesh(),
        scratch_types=[
            pltpu.VMEM((BLK,), jnp.int32),
            pltpu.VMEM((BLK,), jnp.int32),
            pltpu.VMEM((BLK,), jnp.int32),
            pltpu.VMEM((BLK + L,), jnp.int32),
            pltpu.VMEM((BLK,), jnp.int32),
            pltpu.VMEM((BLK, 128), jnp.float32),
            pltpu.VMEM((BLK, 128), jnp.float32),
            pltpu.VMEM((BLK, 128), jnp.float32),
            pltpu.VMEM((BLK, 128), jnp.float32),
            pltpu.VMEM((BLK, L), jnp.float32),
            pltpu.VMEM((K, L), jnp.float32),
            pltpu.VMEM_SHARED((nd_pad, 128), jnp.float32),
            pltpu.SemaphoreType.DMA,
            pltpu.SemaphoreType.DMA,
        ],
        compiler_params=pltpu.CompilerParams(needs_layout_passes=False),
    )
    return f(u2, ef2, src, dstg, dsts, attc, zd)


def _pass_b(u2, exs, src, dsts, zn, u_split):
    K = 8
    n_nodes = (u2.shape[1] if u_split else u2.shape[0])
    n_edges = src.shape[0]
    rows_pt = (-(-n_nodes // NS) + 7) // 8 * 8
    n_pad = rows_pt * NS
    nworkers = NS if u_split else NC * NS
    ept = n_edges // nworkers
    nblk = ept // BLK

    def body(u_ref, ex_ref, src_ref, dsts_ref, zn_ref, on_ref,
             sidx, didxs, us, exb, accn, sem1):
        c = lax.axis_index("c")
        s = lax.axis_index("s")
        r0 = s * rows_pt
        pltpu.sync_copy(zn_ref, accn.at[pl.ds(r0, rows_pt)])
        plsc.subcore_barrier()
        w = s if u_split else (c * NS + s)

        def blk_body(b, carry):
            base = w * ept + b * BLK
            pltpu.sync_copy(src_ref.at[pl.ds(base, BLK)], sidx)
            pltpu.sync_copy(dsts_ref.at[pl.ds(base, BLK)], didxs)
            if u_split:
                cp1 = pltpu.async_copy(u_ref.at[c].at[sidx], us, sem1)
                pltpu.sync_copy(ex_ref.at[c].at[pl.ds(base, BLK)], exb)
            else:
                cp1 = pltpu.async_copy(u_ref.at[sidx], us, sem1)
                pltpu.sync_copy(ex_ref.at[0].at[pl.ds(base, BLK)], exb)
            cp1.wait()

            def edge(i, carry2):
                exv = exb[i, :]
                for k in range(K):
                    sl = pl.ds(k * L, L)
                    us[i, sl] = us[i, sl] * exv
                return carry2

            lax.fori_loop(0, BLK, edge, 0)
            pltpu.sync_copy(us, accn.at[didxs], add=True)
            return carry

        lax.fori_loop(0, nblk, blk_body, 0)
        plsc.subcore_barrier()
        pltpu.sync_copy(accn.at[pl.ds(r0, rows_pt)],
                        on_ref.at[c].at[pl.ds(r0, rows_pt)])

    f = pl.kernel(
        body,
        out_type=jax.ShapeDtypeStruct((NC, n_pad, 128), jnp.float32),
        mesh=_sc_mesh(),
        scratch_types=[
            pltpu.VMEM((BLK,), jnp.int32),
            pltpu.VMEM((BLK,), jnp.int32),
            pltpu.VMEM((BLK, 128), jnp.float32),
            pltpu.VMEM((BLK, L), jnp.float32),
            pltpu.VMEM_SHARED((n_pad, 128), jnp.float32),
            pltpu.SemaphoreType.DMA,
        ],
        compiler_params=pltpu.CompilerParams(needs_layout_passes=False),
    )
    return f(u2, exs, src, dsts, zn)


def _gat_sc(u2, ef2, src, dstg, dsts, attc, zn, zd, u_split):
    exs, den = _pass_a(u2, ef2, src, dstg, dsts, attc, zd, u_split)
    num = _pass_b(u2, exs, src, dsts, zn, u_split)
    return num, den


def _gat3_sc(u3, f3, src, dstg, dsts, att3v, z3):
    n_nodes = u3.shape[0] * 16
    n_edges = src.shape[0]
    epw = n_edges // (NC * NS)
    nblk = epw // BLK
    rows_pt = (-(-(-(-n_nodes // 16) // NS)) + 7) // 8 * 8
    n3_pad = rows_pt * NS

    def body(u_ref, f_ref, src_ref, dstg_ref, dsts_ref, att_ref, z_ref,
             out_ref, u3v, sidx, didxg, didxs, didx8, f3v, wb3, attv, acc):
        c = lax.axis_index("c")
        s = lax.axis_index("s")
        pltpu.sync_copy(u_ref, u3v)
        pltpu.sync_copy(att_ref, attv)
        r0 = s * rows_pt
        pltpu.sync_copy(z_ref, acc.at[pl.ds(r0, rows_pt)])
        iot = lax.iota(jnp.int32, L)
        zer = jnp.zeros((L,), jnp.int32)
        one = zer + 1
        _zero_wbd(wb3, iot)
        plsc.subcore_barrier()
        w = c * NS + s

        def blk_body(b, carry):
            base = w * epw + b * BLK
            pltpu.sync_copy(src_ref.at[pl.ds(base, BLK)], sidx)
            pltpu.sync_copy(dstg_ref.at[pl.ds(base, BLK)], didxg)
            pltpu.sync_copy(dsts_ref.at[pl.ds(base, BLK)], didxs)
            pltpu.sync_copy(
                f_ref.at[pl.ds(lax.div(base, 16), BLK // L)], f3v)
            for j in range(BLK // L):
                sl = pl.ds(j * L, L)
                didx8[sl] = jnp.right_shift(didxs[sl], 4)
            for j in range(BLK // L):
                sl = pl.ds(j * L, L)
                rows = iot + (j * L)
                sv = sidx[sl]
                dv = didxg[sl]
                ds_ = didxs[sl]
                ul = plsc.load_gather(
                    u3v, [jnp.right_shift(sv, 4), (sv & 15) * 8])
                ur = plsc.load_gather(
                    u3v, [jnp.right_shift(dv, 4), (dv & 15) * 8 + 1])
                fv = plsc.load_gather(f3v, [jnp.full((L,), j, jnp.int32),
                                            iot * 8])
                e = ul + ur + fv
                e = jnp.maximum(e, 0.0) + 0.2 * jnp.minimum(e, 0.0)
                al = e * attv[...]
                exv = jnp.exp(jnp.minimum(al, 60.0))
                cols = (ds_ & 15) * 8
                plsc.store_scatter(wb3, [rows, cols], ul * exv)
                plsc.store_scatter(wb3, [rows, cols + 1], exv)
            pltpu.sync_copy(wb3, acc.at[didx8], add=True)
            for j in range(BLK // L):
                sl = pl.ds(j * L, L)
                rows = iot + (j * L)
                cols = (didxs[sl] & 15) * 8
                plsc.store_scatter(wb3, [rows, cols],
                                   jnp.zeros((L,), jnp.float32))
                plsc.store_scatter(wb3, [rows, cols + 1],
                                   jnp.zeros((L,), jnp.float32))
            return carry

        lax.fori_loop(0, nblk, blk_body, 0)
        plsc.subcore_barrier()
        pltpu.sync_copy(acc.at[pl.ds(r0, rows_pt)],
                        out_ref.at[c].at[pl.ds(r0, rows_pt)])

    f = pl.kernel(
        body,
        out_type=jax.ShapeDtypeStruct((NC, n3_pad, 128), jnp.float32),
        mesh=_sc_mesh(),
        scratch_types=[
            pltpu.VMEM((n_nodes // 16, 128), jnp.float32),
            pltpu.VMEM((BLK,), jnp.int32),
            pltpu.VMEM((BLK,), jnp.int32),
            pltpu.VMEM((BLK,), jnp.int32),
            pltpu.VMEM((BLK,), jnp.int32),
            pltpu.VMEM((BLK // L, 128), jnp.float32),
            pltpu.VMEM((BLK, 128), jnp.float32),
            pltpu.VMEM((L,), jnp.float32),
            pltpu.VMEM_SHARED((n3_pad, 128), jnp.float32),
        ],
        compiler_params=pltpu.CompilerParams(needs_layout_passes=False),
    )
    return f(u3, f3, src, dstg, dsts, att3v, z3)


def _mm_halves(xm, Wt, bt, bm):
    m, kin = xm.shape
    nh = Wt.shape[2]

    def kfn(x_ref, w_ref, b_ref, o_ref):
        h = pl.program_id(0)
        o_ref[0] = (jnp.dot(x_ref[...], w_ref[0],
                            preferred_element_type=jnp.float32)
                    + b_ref[h][None, :])

    return pl.pallas_call(
        kfn,
        grid=(2, m // bm),
        in_specs=[
            pl.BlockSpec((bm, kin), lambda h, i: (i, 0)),
            pl.BlockSpec((1, kin, nh), lambda h, i: (h, 0, 0)),
            pl.BlockSpec((2, nh), lambda h, i: (0, 0)),
        ],
        out_specs=pl.BlockSpec((1, bm, nh), lambda h, i: (h, i, 0)),
        out_shape=jax.ShapeDtypeStruct((2, m, nh), jnp.float32),
    )(xm, Wt, bt)


def _mm_plain(xm, w, b, bm):
    m, kin = xm.shape
    nh = w.shape[1]

    def kfn(x_ref, w_ref, b_ref, o_ref):
        o_ref[...] = (jnp.dot(x_ref[...], w_ref[...],
                              preferred_element_type=jnp.float32)
                      + b_ref[...][None, :])

    return pl.pallas_call(
        kfn,
        grid=(m // bm,),
        in_specs=[
            pl.BlockSpec((bm, kin), lambda i: (i, 0)),
            pl.BlockSpec((kin, nh), lambda i: (0, 0)),
            pl.BlockSpec((nh,), lambda i: (0,)),
        ],
        out_specs=pl.BlockSpec((bm, nh), lambda i: (i, 0)),
        out_shape=jax.ShapeDtypeStruct((m, nh), jnp.float32),
    )(xm, w, b)


_BN_INV = (1.0 + 1e-5) ** -0.5


def _epilogue1(num, den, biasp, gp, bep, bn):
    n_nodes = 10000

    def kfn(n_ref, d_ref, b_ref, g_ref, e_ref, o_ref):
        outs = []
        for c in range(2):
            dt = jnp.concatenate([d_ref[c]] * 8, axis=1)
            z = n_ref[c] / (dt + 1e-16) + b_ref[c][None, :]
            z = g_ref[c][None, :] * z * _BN_INV + e_ref[c][None, :]
            z = jnp.where(z > 0, z, jnp.exp(jnp.minimum(z, 0.0)) - 1.0)
            outs.append(z)
        o_ref[...] = jnp.concatenate(outs, axis=1)

    return pl.pallas_call(
        kfn,
        grid=(n_nodes // bn,),
        in_specs=[
            pl.BlockSpec((2, bn, 128), lambda i: (0, i, 0)),
            pl.BlockSpec((2, bn, 16), lambda i: (0, i, 0)),
            pl.BlockSpec((2, 128), lambda i: (0, 0)),
            pl.BlockSpec((2, 128), lambda i: (0, 0)),
            pl.BlockSpec((2, 128), lambda i: (0, 0)),
        ],
        out_specs=pl.BlockSpec((bn, 256), lambda i: (i, 0)),
        out_shape=jax.ShapeDtypeStruct((n_nodes, 256), jnp.float32),
    )(num, den, biasp, gp, bep)


def _epilogue2(num, den, biasp, gp, bep, bn):
    n_nodes = 10000

    def kfn(n_ref, d_ref, b_ref, g_ref, e_ref, o_ref):
        nm = n_ref[0] + n_ref[1]
        dn = d_ref[0] + d_ref[1]
        dt = jnp.concatenate([dn] * 8, axis=1)
        z = nm / (dt + 1e-16) + b_ref[...][None, :]
        z = g_ref[...][None, :] * z * _BN_INV + e_ref[...][None, :]
        z = jnp.where(z > 0, z, jnp.exp(jnp.minimum(z, 0.0)) - 1.0)
        o_ref[...] = z

    return pl.pallas_call(
        kfn,
        grid=(n_nodes // bn,),
        in_specs=[
            pl.BlockSpec((2, bn, 128), lambda i: (0, i, 0)),
            pl.BlockSpec((2, bn, 16), lambda i: (0, i, 0)),
            pl.BlockSpec((128,), lambda i: (0,)),
            pl.BlockSpec((128,), lambda i: (0,)),
            pl.BlockSpec((128,), lambda i: (0,)),
        ],
        out_specs=pl.BlockSpec((bn, 128), lambda i: (i, 0)),
        out_shape=jax.ShapeDtypeStruct((n_nodes, 128), jnp.float32),
    )(num, den, biasp, gp, bep)


def _final3(p3, bias3, bn):
    n_nodes = 10000

    def kfn(a_ref, b_ref, o_ref):
        num = a_ref[0][:, 0:1] + a_ref[1][:, 0:1]
        den = a_ref[0][:, 1:2] + a_ref[1][:, 1:2]
        o_ref[...] = num / (den + 1e-16) + b_ref[...]

    return pl.pallas_call(
        kfn,
        grid=(n_nodes // bn,),
        in_specs=[
            pl.BlockSpec((2, bn, 8), lambda i: (0, i, 0)),
            pl.BlockSpec((1, 1), lambda i: (0, 0)),
        ],
        out_specs=pl.BlockSpec((bn, 1), lambda i: (i, 0)),
        out_shape=jax.ShapeDtypeStruct((n_nodes, 1), jnp.float32),
    )(p3, bias3)


def kernel(x, edge_index, edge_attr, W1, b1, We1, att1, bias1, g1, be1,
           W2, b2, We2, att2, bias2, g2, be2,
           W3l, b3l, W3r, b3r, We3, att3, bias3):
    n_nodes = x.shape[0]
    n_edges = edge_index.shape[1]

    e_pad = -(-n_edges // (NC * NS * BLK)) * (NC * NS * BLK)
    npad_e = e_pad - n_edges
    src = jnp.concatenate(
        [edge_index[0].astype(jnp.int32), jnp.zeros((npad_e,), jnp.int32)])
    dstg = jnp.concatenate(
        [edge_index[1].astype(jnp.int32), jnp.zeros((npad_e,), jnp.int32)])
    pad_node = 10080
    dsts = jnp.concatenate(
        [edge_index[1].astype(jnp.int32),
         jnp.full((npad_e,), pad_node, jnp.int32)])
    ea_p = jnp.concatenate(
        [edge_attr, jnp.zeros((npad_e, edge_attr.shape[1]), jnp.float32)])

    K1, K2 = 8, 8
    Wt1 = jnp.transpose(_perm_cols_split(W1, K1), (1, 0, 2))
    bt1 = _perm_cols_split(b1, K1)
    Wet1 = jnp.transpose(_perm_cols_split(We1, K1), (1, 0, 2))
    attc1 = _perm_cols_split(att1.reshape(-1), K1).reshape(2, K1, L)
    biasp1 = _perm_cols_split(bias1, K1)
    gp1 = _perm_cols_split(g1, K1)
    bep1 = _perm_cols_split(be1, K1)
    perm1 = _perm_cols_split(jnp.arange(256), K1).reshape(-1)

    W2p = _perm_cols_flat(W2[perm1, :], K2)
    b2p = _perm_cols_flat(b2, K2)
    We2p = _perm_cols_flat(We2, K2)
    attc2 = _perm_cols_flat(att2.reshape(-1), K2).reshape(K2, L)
    biasp2 = _perm_cols_flat(bias2, K2)
    gp2 = _perm_cols_flat(g2, K2)
    bep2 = _perm_cols_flat(be2, K2)
    perm2 = _perm_cols_flat(jnp.arange(128), K2)

    W38 = jnp.concatenate(
        [W3l[perm2, :], W3r[perm2, :], jnp.zeros((128, 6), jnp.float32)],
        axis=1)
    b38 = jnp.concatenate([b3l, b3r, jnp.zeros((6,), jnp.float32)])
    We38 = jnp.concatenate([We3, jnp.zeros((16, 7), jnp.float32)], axis=1)
    b08 = jnp.zeros((8,), jnp.float32)
    att3v = jnp.full((L,), att3[0, 0], jnp.float32)
    zb = jnp.zeros((2, 128), jnp.float32)

    rows_pt = (-(-n_nodes // NS) + 7) // 8 * 8
    rowsd_pt = (-(-(rows_pt * NS // 8) // NS) + 7) // 8 * 8
    rows3_pt = (-(-(-(-n_nodes // 16) // NS)) + 7) // 8 * 8
    zn = jnp.zeros((rows_pt, 128), jnp.float32)
    zd = jnp.zeros((rowsd_pt, 128), jnp.float32)
    z3 = jnp.zeros((rows3_pt, 128), jnp.float32)

    u1 = _mm_halves(x, Wt1, bt1, 2000)
    ef1 = _mm_halves(ea_p, Wet1, zb, 4096)
    num1, den1 = _gat_sc(u1, ef1, src, dstg, dsts, attc1, zn, zd, True)
    den1r = den1.reshape(2, -1, 16)
    h1 = _epilogue1(num1, den1r, biasp1, gp1, bep1, 2000)

    u2 = _mm_plain(h1, W2p, b2p, 2000)
    ef2 = _mm_plain(ea_p, We2p, b2p * 0.0, 4096)
    num2, den2 = _gat_sc(u2, ef2, src, dstg, dsts, attc2, zn, zd, False)
    den2r = den2.reshape(2, -1, 16)
    h2 = _epilogue2(num2, den2r, biasp2, gp2, bep2, 2000)

    u3 = _mm_plain(h2, W38, b38, 2000).reshape(-1, 128)
    f3 = _mm_plain(ea_p, We38, b08, 4096).reshape(-1, 128)
    acc3 = _gat3_sc(u3, f3, src, dstg, dsts, att3v, z3)
    p3 = acc3.reshape(2, -1, 8)
    return _final3(p3, bias3.reshape(1, 1), 2000)


# speedup vs baseline: 19.7091x; 1.3143x over previous
import functools

import jax
import jax.numpy as jnp
from jax import lax
from jax.experimental import pallas as pl
from jax.experimental.pallas import tpu as pltpu
from jax.experimental.pallas import tpu_sc as plsc

NC = 2
NS = 16
L = 16
BLK = 128


def _perm_cols_split(a, K):
    lead = a.shape[:-1]
    r = a.reshape(*lead, 2, 4, 4, K)
    r = jnp.moveaxis(r, -1, -3)
    return r.reshape(*lead, 2, 16 * K)


def _perm_cols_flat(a, K):
    lead = a.shape[:-1]
    r = a.reshape(*lead, 8, 2, K)
    r = jnp.moveaxis(r, -1, -3)
    return r.reshape(*lead, 16 * K)


def _sc_mesh():
    return plsc.VectorSubcoreMesh(core_axis_name="c", subcore_axis_name="s",
                                  num_cores=NC, num_subcores=NS)


def _zero_wbd(wbd, iot):
    def zrow(t, carry):
        cols = (t & 7) * L + iot
        row = jnp.full((L,), t >> 3, jnp.int32)
        plsc.store_scatter(wbd, [row, cols], jnp.zeros((L,), jnp.float32))
        return carry
    lax.fori_loop(0, BLK * 8, zrow, 0)


def _pass_a(u2, ef2, src, dstg, dsts, attc, zd, u_split):
    K = 8
    n_nodes = (u2.shape[1] if u_split else u2.shape[0])
    n_edges = src.shape[0]
    rows_pt = (-(-n_nodes // NS) + 7) // 8 * 8
    n_pad = rows_pt * NS
    rowsd_pt = (-(-(n_pad // 8) // NS) + 7) // 8 * 8
    nd_pad = rowsd_pt * NS
    nworkers = NS if u_split else NC * NS
    ept = n_edges // nworkers
    nblk = ept // BLK
    nshuf = 2 if u_split else 1

    def body(u_ref, ef_ref, src_ref, dstg_ref, dsts_ref, att_ref, zd_ref,
             ox_ref, od_ref,
             sidx, didxg, didxs, didxp, didx8, us, ud, ef, wbd, exb, attv,
             accd, sem1, sem2):
        c = lax.axis_index("c")
        s = lax.axis_index("s")
        pltpu.sync_copy(att_ref.at[c] if u_split else att_ref, attv)
        rd0 = s * rowsd_pt
        pltpu.sync_copy(zd_ref, accd.at[pl.ds(rd0, rowsd_pt)])
        iot = lax.iota(jnp.int32, L)
        _zero_wbd(wbd, iot)
        plsc.subcore_barrier()
        shufs = [iot ^ 1, iot ^ 2][:nshuf]
        w = s if u_split else (c * NS + s)

        def blk_body(b, carry):
            base = w * ept + b * BLK
            pltpu.sync_copy(src_ref.at[pl.ds(base, BLK)], sidx)
            pltpu.sync_copy(dstg_ref.at[pl.ds(base, BLK)], didxg)
            pltpu.sync_copy(dsts_ref.at[pl.ds(base, BLK)], didxs)
            pltpu.sync_copy(dsts_ref.at[pl.ds(base, BLK)],
                            didxp.at[pl.ds(0, BLK)])
            if u_split:
                cp1 = pltpu.async_copy(u_ref.at[c].at[sidx], us, sem1)
                cp2 = pltpu.async_copy(u_ref.at[c].at[didxg], ud, sem2)
                pltpu.sync_copy(ef_ref.at[c].at[pl.ds(base, BLK)], ef)
            else:
                cp1 = pltpu.async_copy(u_ref.at[sidx], us, sem1)
                cp2 = pltpu.async_copy(u_ref.at[didxg], ud, sem2)
                pltpu.sync_copy(ef_ref.at[pl.ds(base, BLK)], ef)
            for j in range(BLK // L):
                sl = pl.ds(j * L, L)
                didx8[sl] = jnp.right_shift(didxs[sl], 3)
            cp1.wait()
            cp2.wait()

            @plsc.parallel_loop(0, BLK, unroll=2)
            def edge(i):
                accv = jnp.zeros((L,), jnp.float32)
                for k in range(K):
                    sl = pl.ds(k * L, L)
                    e = us[i, sl] + ud[i, sl] + ef[i, sl]
                    e = jnp.maximum(e, 0.0) + 0.2 * jnp.minimum(e, 0.0)
                    accv = accv + e * attv[k]
                for sh in shufs:
                    accv = accv + jnp.take_along_axis(accv, sh, axis=0)
                exv = jnp.exp(jnp.minimum(accv, 60.0))
                exb[i, :] = exv
                d = didxp[pl.ds(i, L)][0]
                cols = (d & 7) * L + iot
                row = jnp.full((L,), i, jnp.int32)
                plsc.store_scatter(wbd, [row, cols], exv)
            pltpu.sync_copy(wbd, accd.at[didx8], add=True)
            if u_split:
                pltpu.sync_copy(exb, ox_ref.at[c].at[pl.ds(base, BLK)])
            else:
                pltpu.sync_copy(exb, ox_ref.at[0].at[pl.ds(base, BLK)])

            @plsc.parallel_loop(0, BLK, unroll=4)
            def zedge(i):
                d = didxp[pl.ds(i, L)][0]
                cols = (d & 7) * L + iot
                row = jnp.full((L,), i, jnp.int32)
                plsc.store_scatter(wbd, [row, cols],
                                   jnp.zeros((L,), jnp.float32))
            return carry

        lax.fori_loop(0, nblk, blk_body, 0)
        plsc.subcore_barrier()
        pltpu.sync_copy(accd.at[pl.ds(rd0, rowsd_pt)],
                        od_ref.at[c].at[pl.ds(rd0, rowsd_pt)])

    n_ex = NC if u_split else 1
    f = pl.kernel(
        body,
        out_type=(
            jax.ShapeDtypeStruct((n_ex, n_edges, L), jnp.float32),
            jax.ShapeDtypeStruct((NC, nd_pad, 128), jnp.float32),
        ),
        mesh=_sc_mesh(),
        scratch_types=[
            pltpu.VMEM((BLK,), jnp.int32),
            pltpu.VMEM((BLK,), jnp.int32),
            pltpu.VMEM((BLK,), jnp.int32),
            pltpu.VMEM((BLK + L,), jnp.int32),
            pltpu.VMEM((BLK,), jnp.int32),
            pltpu.VMEM((BLK, 128), jnp.float32),
            pltpu.VMEM((BLK, 128), jnp.float32),
            pltpu.VMEM((BLK, 128), jnp.float32),
            pltpu.VMEM((BLK, 128), jnp.float32),
            pltpu.VMEM((BLK, L), jnp.float32),
            pltpu.VMEM((K, L), jnp.float32),
            pltpu.VMEM_SHARED((nd_pad, 128), jnp.float32),
            pltpu.SemaphoreType.DMA,
            pltpu.SemaphoreType.DMA,
        ],
        compiler_params=pltpu.CompilerParams(needs_layout_passes=False),
    )
    return f(u2, ef2, src, dstg, dsts, attc, zd)


def _pass_b(u2, exs, src, dsts, zn, u_split):
    K = 8
    n_nodes = (u2.shape[1] if u_split else u2.shape[0])
    n_edges = src.shape[0]
    rows_pt = (-(-n_nodes // NS) + 7) // 8 * 8
    n_pad = rows_pt * NS
    nworkers = NS if u_split else NC * NS
    ept = n_edges // nworkers
    nblk = ept // BLK

    def body(u_ref, ex_ref, src_ref, dsts_ref, zn_ref, on_ref,
             sidx, didxs, us, exb, accn, sem1):
        c = lax.axis_index("c")
        s = lax.axis_index("s")
        r0 = s * rows_pt
        pltpu.sync_copy(zn_ref, accn.at[pl.ds(r0, rows_pt)])
        plsc.subcore_barrier()
        w = s if u_split else (c * NS + s)

        def blk_body(b, carry):
            base = w * ept + b * BLK
            pltpu.sync_copy(src_ref.at[pl.ds(base, BLK)], sidx)
            pltpu.sync_copy(dsts_ref.at[pl.ds(base, BLK)], didxs)
            if u_split:
                cp1 = pltpu.async_copy(u_ref.at[c].at[sidx], us, sem1)
                pltpu.sync_copy(ex_ref.at[c].at[pl.ds(base, BLK)], exb)
            else:
                cp1 = pltpu.async_copy(u_ref.at[sidx], us, sem1)
                pltpu.sync_copy(ex_ref.at[0].at[pl.ds(base, BLK)], exb)
            cp1.wait()

            @plsc.parallel_loop(0, BLK, unroll=2)
            def edge(i):
                exv = exb[i, :]
                for k in range(K):
                    sl = pl.ds(k * L, L)
                    us[i, sl] = us[i, sl] * exv
            pltpu.sync_copy(us, accn.at[didxs], add=True)
            return carry

        lax.fori_loop(0, nblk, blk_body, 0)
        plsc.subcore_barrier()
        pltpu.sync_copy(accn.at[pl.ds(r0, rows_pt)],
                        on_ref.at[c].at[pl.ds(r0, rows_pt)])

    f = pl.kernel(
        body,
        out_type=jax.ShapeDtypeStruct((NC, n_pad, 128), jnp.float32),
        mesh=_sc_mesh(),
        scratch_types=[
            pltpu.VMEM((BLK,), jnp.int32),
            pltpu.VMEM((BLK,), jnp.int32),
            pltpu.VMEM((BLK, 128), jnp.float32),
            pltpu.VMEM((BLK, L), jnp.float32),
            pltpu.VMEM_SHARED((n_pad, 128), jnp.float32),
            pltpu.SemaphoreType.DMA,
        ],
        compiler_params=pltpu.CompilerParams(needs_layout_passes=False),
    )
    return f(u2, exs, src, dsts, zn)


def _gat_sc(u2, ef2, src, dstg, dsts, attc, zn, zd, u_split):
    exs, den = _pass_a(u2, ef2, src, dstg, dsts, attc, zd, u_split)
    num = _pass_b(u2, exs, src, dsts, zn, u_split)
    return num, den


def _gat3_sc(u3, f3, src, dstg, dsts, att3v, z3):
    n_nodes = u3.shape[0] * 16
    n_edges = src.shape[0]
    epw = n_edges // (NC * NS)
    nblk = epw // BLK
    rows_pt = (-(-(-(-n_nodes // 16) // NS)) + 7) // 8 * 8
    n3_pad = rows_pt * NS

    def body(u_ref, f_ref, src_ref, dstg_ref, dsts_ref, att_ref, z_ref,
             out_ref, u3v, sidx, didxg, didxs, didx8, f3v, wb3, attv, acc):
        c = lax.axis_index("c")
        s = lax.axis_index("s")
        pltpu.sync_copy(u_ref, u3v)
        pltpu.sync_copy(att_ref, attv)
        r0 = s * rows_pt
        pltpu.sync_copy(z_ref, acc.at[pl.ds(r0, rows_pt)])
        iot = lax.iota(jnp.int32, L)
        zer = jnp.zeros((L,), jnp.int32)
        one = zer + 1
        _zero_wbd(wb3, iot)
        plsc.subcore_barrier()
        w = c * NS + s

        def blk_body(b, carry):
            base = w * epw + b * BLK
            pltpu.sync_copy(src_ref.at[pl.ds(base, BLK)], sidx)
            pltpu.sync_copy(dstg_ref.at[pl.ds(base, BLK)], didxg)
            pltpu.sync_copy(dsts_ref.at[pl.ds(base, BLK)], didxs)
            pltpu.sync_copy(
                f_ref.at[pl.ds(lax.div(base, 16), BLK // L)], f3v)
            for j in range(BLK // L):
                sl = pl.ds(j * L, L)
                didx8[sl] = jnp.right_shift(didxs[sl], 4)
            for j in range(BLK // L):
                sl = pl.ds(j * L, L)
                rows = iot + (j * L)
                sv = sidx[sl]
                dv = didxg[sl]
                ds_ = didxs[sl]
                ul = plsc.load_gather(
                    u3v, [jnp.right_shift(sv, 4), (sv & 15) * 8])
                ur = plsc.load_gather(
                    u3v, [jnp.right_shift(dv, 4), (dv & 15) * 8 + 1])
                fv = plsc.load_gather(f3v, [jnp.full((L,), j, jnp.int32),
                                            iot * 8])
                e = ul + ur + fv
                e = jnp.maximum(e, 0.0) + 0.2 * jnp.minimum(e, 0.0)
                al = e * attv[...]
                exv = jnp.exp(jnp.minimum(al, 60.0))
                cols = (ds_ & 15) * 8
                plsc.store_scatter(wb3, [rows, cols], ul * exv)
                plsc.store_scatter(wb3, [rows, cols + 1], exv)
            pltpu.sync_copy(wb3, acc.at[didx8], add=True)
            for j in range(BLK // L):
                sl = pl.ds(j * L, L)
                rows = iot + (j * L)
                cols = (didxs[sl] & 15) * 8
                plsc.store_scatter(wb3, [rows, cols],
                                   jnp.zeros((L,), jnp.float32))
                plsc.store_scatter(wb3, [rows, cols + 1],
                                   jnp.zeros((L,), jnp.float32))
            return carry

        lax.fori_loop(0, nblk, blk_body, 0)
        plsc.subcore_barrier()
        pltpu.sync_copy(acc.at[pl.ds(r0, rows_pt)],
                        out_ref.at[c].at[pl.ds(r0, rows_pt)])

    f = pl.kernel(
        body,
        out_type=jax.ShapeDtypeStruct((NC, n3_pad, 128), jnp.float32),
        mesh=_sc_mesh(),
        scratch_types=[
            pltpu.VMEM((n_nodes // 16, 128), jnp.float32),
            pltpu.VMEM((BLK,), jnp.int32),
            pltpu.VMEM((BLK,), jnp.int32),
            pltpu.VMEM((BLK,), jnp.int32),
            pltpu.VMEM((BLK,), jnp.int32),
            pltpu.VMEM((BLK // L, 128), jnp.float32),
            pltpu.VMEM((BLK, 128), jnp.float32),
            pltpu.VMEM((L,), jnp.float32),
            pltpu.VMEM_SHARED((n3_pad, 128), jnp.float32),
        ],
        compiler_params=pltpu.CompilerParams(needs_layout_passes=False),
    )
    return f(u3, f3, src, dstg, dsts, att3v, z3)


def _mm_halves(xm, Wt, bt, bm):
    m, kin = xm.shape
    nh = Wt.shape[2]

    def kfn(x_ref, w_ref, b_ref, o_ref):
        h = pl.program_id(0)
        o_ref[0] = (jnp.dot(x_ref[...], w_ref[0],
                            preferred_element_type=jnp.float32)
                    + b_ref[h][None, :])

    return pl.pallas_call(
        kfn,
        grid=(2, m // bm),
        in_specs=[
            pl.BlockSpec((bm, kin), lambda h, i: (i, 0)),
            pl.BlockSpec((1, kin, nh), lambda h, i: (h, 0, 0)),
            pl.BlockSpec((2, nh), lambda h, i: (0, 0)),
        ],
        out_specs=pl.BlockSpec((1, bm, nh), lambda h, i: (h, i, 0)),
        out_shape=jax.ShapeDtypeStruct((2, m, nh), jnp.float32),
    )(xm, Wt, bt)


def _mm_plain(xm, w, b, bm):
    m, kin = xm.shape
    nh = w.shape[1]

    def kfn(x_ref, w_ref, b_ref, o_ref):
        o_ref[...] = (jnp.dot(x_ref[...], w_ref[...],
                              preferred_element_type=jnp.float32)
                      + b_ref[...][None, :])

    return pl.pallas_call(
        kfn,
        grid=(m // bm,),
        in_specs=[
            pl.BlockSpec((bm, kin), lambda i: (i, 0)),
            pl.BlockSpec((kin, nh), lambda i: (0, 0)),
            pl.BlockSpec((nh,), lambda i: (0,)),
        ],
        out_specs=pl.BlockSpec((bm, nh), lambda i: (i, 0)),
        out_shape=jax.ShapeDtypeStruct((m, nh), jnp.float32),
    )(xm, w, b)


_BN_INV = (1.0 + 1e-5) ** -0.5


def _epilogue1(num, den, biasp, gp, bep, bn):
    n_nodes = 10000

    def kfn(n_ref, d_ref, b_ref, g_ref, e_ref, o_ref):
        outs = []
        for c in range(2):
            dt = jnp.concatenate([d_ref[c]] * 8, axis=1)
            z = n_ref[c] / (dt + 1e-16) + b_ref[c][None, :]
            z = g_ref[c][None, :] * z * _BN_INV + e_ref[c][None, :]
            z = jnp.where(z > 0, z, jnp.exp(jnp.minimum(z, 0.0)) - 1.0)
            outs.append(z)
        o_ref[...] = jnp.concatenate(outs, axis=1)

    return pl.pallas_call(
        kfn,
        grid=(n_nodes // bn,),
        in_specs=[
            pl.BlockSpec((2, bn, 128), lambda i: (0, i, 0)),
            pl.BlockSpec((2, bn, 16), lambda i: (0, i, 0)),
            pl.BlockSpec((2, 128), lambda i: (0, 0)),
            pl.BlockSpec((2, 128), lambda i: (0, 0)),
            pl.BlockSpec((2, 128), lambda i: (0, 0)),
        ],
        out_specs=pl.BlockSpec((bn, 256), lambda i: (i, 0)),
        out_shape=jax.ShapeDtypeStruct((n_nodes, 256), jnp.float32),
    )(num, den, biasp, gp, bep)


def _epilogue2(num, den, biasp, gp, bep, bn):
    n_nodes = 10000

    def kfn(n_ref, d_ref, b_ref, g_ref, e_ref, o_ref):
        nm = n_ref[0] + n_ref[1]
        dn = d_ref[0] + d_ref[1]
        dt = jnp.concatenate([dn] * 8, axis=1)
        z = nm / (dt + 1e-16) + b_ref[...][None, :]
        z = g_ref[...][None, :] * z * _BN_INV + e_ref[...][None, :]
        z = jnp.where(z > 0, z, jnp.exp(jnp.minimum(z, 0.0)) - 1.0)
        o_ref[...] = z

    return pl.pallas_call(
        kfn,
        grid=(n_nodes // bn,),
        in_specs=[
            pl.BlockSpec((2, bn, 128), lambda i: (0, i, 0)),
            pl.BlockSpec((2, bn, 16), lambda i: (0, i, 0)),
            pl.BlockSpec((128,), lambda i: (0,)),
            pl.BlockSpec((128,), lambda i: (0,)),
            pl.BlockSpec((128,), lambda i: (0,)),
        ],
        out_specs=pl.BlockSpec((bn, 128), lambda i: (i, 0)),
        out_shape=jax.ShapeDtypeStruct((n_nodes, 128), jnp.float32),
    )(num, den, biasp, gp, bep)


def _final3(p3, bias3, bn):
    n_nodes = 10000

    def kfn(a_ref, b_ref, o_ref):
        num = a_ref[0][:, 0:1] + a_ref[1][:, 0:1]
        den = a_ref[0][:, 1:2] + a_ref[1][:, 1:2]
        o_ref[...] = num / (den + 1e-16) + b_ref[...]

    return pl.pallas_call(
        kfn,
        grid=(n_nodes // bn,),
        in_specs=[
            pl.BlockSpec((2, bn, 8), lambda i: (0, i, 0)),
            pl.BlockSpec((1, 1), lambda i: (0, 0)),
        ],
        out_specs=pl.BlockSpec((bn, 1), lambda i: (i, 0)),
        out_shape=jax.ShapeDtypeStruct((n_nodes, 1), jnp.float32),
    )(p3, bias3)


def kernel(x, edge_index, edge_attr, W1, b1, We1, att1, bias1, g1, be1,
           W2, b2, We2, att2, bias2, g2, be2,
           W3l, b3l, W3r, b3r, We3, att3, bias3):
    n_nodes = x.shape[0]
    n_edges = edge_index.shape[1]

    e_pad = -(-n_edges // (NC * NS * BLK)) * (NC * NS * BLK)
    npad_e = e_pad - n_edges
    src = jnp.concatenate(
        [edge_index[0].astype(jnp.int32), jnp.zeros((npad_e,), jnp.int32)])
    dstg = jnp.concatenate(
        [edge_index[1].astype(jnp.int32), jnp.zeros((npad_e,), jnp.int32)])
    pad_node = 10080
    dsts = jnp.concatenate(
        [edge_index[1].astype(jnp.int32),
         jnp.full((npad_e,), pad_node, jnp.int32)])
    ea_p = jnp.concatenate(
        [edge_attr, jnp.zeros((npad_e, edge_attr.shape[1]), jnp.float32)])

    K1, K2 = 8, 8
    Wt1 = jnp.transpose(_perm_cols_split(W1, K1), (1, 0, 2))
    bt1 = _perm_cols_split(b1, K1)
    Wet1 = jnp.transpose(_perm_cols_split(We1, K1), (1, 0, 2))
    attc1 = _perm_cols_split(att1.reshape(-1), K1).reshape(2, K1, L)
    biasp1 = _perm_cols_split(bias1, K1)
    gp1 = _perm_cols_split(g1, K1)
    bep1 = _perm_cols_split(be1, K1)
    perm1 = _perm_cols_split(jnp.arange(256), K1).reshape(-1)

    W2p = _perm_cols_flat(W2[perm1, :], K2)
    b2p = _perm_cols_flat(b2, K2)
    We2p = _perm_cols_flat(We2, K2)
    attc2 = _perm_cols_flat(att2.reshape(-1), K2).reshape(K2, L)
    biasp2 = _perm_cols_flat(bias2, K2)
    gp2 = _perm_cols_flat(g2, K2)
    bep2 = _perm_cols_flat(be2, K2)
    perm2 = _perm_cols_flat(jnp.arange(128), K2)

    W38 = jnp.concatenate(
        [W3l[perm2, :], W3r[perm2, :], jnp.zeros((128, 6), jnp.float32)],
        axis=1)
    b38 = jnp.concatenate([b3l, b3r, jnp.zeros((6,), jnp.float32)])
    We38 = jnp.concatenate([We3, jnp.zeros((16, 7), jnp.float32)], axis=1)
    b08 = jnp.zeros((8,), jnp.float32)
    att3v = jnp.full((L,), att3[0, 0], jnp.float32)
    zb = jnp.zeros((2, 128), jnp.float32)

    rows_pt = (-(-n_nodes // NS) + 7) // 8 * 8
    rowsd_pt = (-(-(rows_pt * NS // 8) // NS) + 7) // 8 * 8
    rows3_pt = (-(-(-(-n_nodes // 16) // NS)) + 7) // 8 * 8
    zn = jnp.zeros((rows_pt, 128), jnp.float32)
    zd = jnp.zeros((rowsd_pt, 128), jnp.float32)
    z3 = jnp.zeros((rows3_pt, 128), jnp.float32)

    u1 = _mm_halves(x, Wt1, bt1, 2000)
    ef1 = _mm_halves(ea_p, Wet1, zb, 4096)
    num1, den1 = _gat_sc(u1, ef1, src, dstg, dsts, attc1, zn, zd, True)
    den1r = den1.reshape(2, -1, 16)
    h1 = _epilogue1(num1, den1r, biasp1, gp1, bep1, 2000)

    u2 = _mm_plain(h1, W2p, b2p, 2000)
    ef2 = _mm_plain(ea_p, We2p, b2p * 0.0, 4096)
    num2, den2 = _gat_sc(u2, ef2, src, dstg, dsts, attc2, zn, zd, False)
    den2r = den2.reshape(2, -1, 16)
    h2 = _epilogue2(num2, den2r, biasp2, gp2, bep2, 2000)

    u3 = _mm_plain(h2, W38, b38, 2000).reshape(-1, 128)
    f3 = _mm_plain(ea_p, We38, b08, 4096).reshape(-1, 128)
    acc3 = _gat3_sc(u3, f3, src, dstg, dsts, att3v, z3)
    p3 = acc3.reshape(2, -1, 8)
    return _final3(p3, bias3.reshape(1, 1), 2000)
